# baseline (speedup 1.0000x reference)
import functools

import jax
import jax.numpy as jnp
from jax import lax
from jax.experimental import pallas as pl
from jax.experimental.pallas import tpu as pltpu
from jax.experimental.pallas import tpu_sc as plsc


def _sc_gather(emb, t):
    info = plsc.get_sparse_core_info()
    nw = info.num_cores * info.num_subcores
    B = t.shape[0]
    D = emb.shape[1]
    b_per_w = B // nw
    mesh = plsc.VectorSubcoreMesh(core_axis_name="c", subcore_axis_name="s")

    @functools.partial(
        pl.kernel,
        mesh=mesh,
        out_type=jax.ShapeDtypeStruct((B, D), jnp.float32),
        scratch_types=[
            pltpu.VMEM((b_per_w,), jnp.int32),
            pltpu.VMEM((b_per_w, D), jnp.float32),
            pltpu.SemaphoreType.DMA,
        ],
    )
    def gather(emb_hbm, t_hbm, out_hbm, idx_v, rows_v, sem):
        wid = lax.axis_index("s") * info.num_cores + lax.axis_index("c")
        base = wid * b_per_w
        pltpu.sync_copy(t_hbm.at[pl.ds(base, b_per_w)], idx_v)
        pltpu.async_copy(emb_hbm.at[idx_v], rows_v, sem).wait()
        pltpu.sync_copy(rows_v, out_hbm.at[pl.ds(base, b_per_w)])

    return gather(emb, t)


def _stream_body(e_ref, w_ref, b_ref, x_ref, out_ref, te_ref):
    @pl.when(pl.program_id(1) == 0)
    def _compute_te():
        te_ref[...] = (
            lax.dot_general(
                e_ref[...].astype(jnp.bfloat16), w_ref[...],
                (((1,), (1,)), ((), ())),
                preferred_element_type=jnp.float32,
            )
            + b_ref[...]
        )

    out_ref[...] = x_ref[...] + te_ref[...][None]


def _stream_body_aliased(e_ref, w_ref, b_ref, x_ref, tmp_ref, out_ref, te_ref):
    _stream_body(e_ref, w_ref, b_ref, x_ref, out_ref, te_ref)


_CHUNK = 128
_BHB = 16
_T0 = 256


def _stream_a(e0, Wb, b2, xr):
    bh, T, N = xr.shape
    return pl.pallas_call(
        _stream_body,
        grid=(_T0 // _CHUNK, bh // _BHB),
        in_specs=[
            pl.BlockSpec((_CHUNK, N), lambda i, j: (i, 0)),
            pl.BlockSpec((N, N), lambda i, j: (0, 0)),
            pl.BlockSpec((1, N), lambda i, j: (0, 0)),
            pl.BlockSpec((_BHB, _CHUNK, N), lambda i, j: (j, i, 0)),
        ],
        out_specs=pl.BlockSpec((_BHB, _CHUNK, N), lambda i, j: (j, i, 0)),
        out_shape=jax.ShapeDtypeStruct((bh, T, N), jnp.float32),
        scratch_shapes=[pltpu.VMEM((_CHUNK, N), jnp.float32)],
    )(e0, Wb, b2, xr)


def _stream_b(e1, Wb, b2, xr, tmp):
    bh, T, N = xr.shape
    s0 = _T0 // _CHUNK
    return pl.pallas_call(
        _stream_body_aliased,
        grid=((T - _T0) // _CHUNK, bh // _BHB),
        in_specs=[
            pl.BlockSpec((_CHUNK, N), lambda i, j: (i, 0)),
            pl.BlockSpec((N, N), lambda i, j: (0, 0)),
            pl.BlockSpec((1, N), lambda i, j: (0, 0)),
            pl.BlockSpec((_BHB, _CHUNK, N), lambda i, j: (j, i + s0, 0)),
            pl.BlockSpec(memory_space=pltpu.MemorySpace.HBM),
        ],
        out_specs=pl.BlockSpec((_BHB, _CHUNK, N), lambda i, j: (j, i + s0, 0)),
        out_shape=jax.ShapeDtypeStruct((bh, T, N), jnp.float32),
        scratch_shapes=[pltpu.VMEM((_CHUNK, N), jnp.float32)],
        input_output_aliases={4: 0},
    )(e1, Wb, b2, xr, tmp)


def kernel(x, t, emb, W, b):
    B2, H, T, N = x.shape
    bh = B2 * H
    xr = x.reshape(bh, T, N)
    e0 = _sc_gather(emb, t[:_T0])
    e1 = _sc_gather(emb, t[_T0:])
    Wb = W.astype(jnp.bfloat16)
    b2 = b.reshape(1, N)
    tmp = _stream_a(e0, Wb, b2, xr)
    out = _stream_b(e1, Wb, b2, xr, tmp)
    return out.reshape(B2, H, T, N)


# speedup vs baseline: 1.0012x; 1.0012x over previous
import functools

import jax
import jax.numpy as jnp
from jax import lax
from jax.experimental import pallas as pl
from jax.experimental.pallas import tpu as pltpu
from jax.experimental.pallas import tpu_sc as plsc


def _sc_gather(emb, t, offset, count):
    info = plsc.get_sparse_core_info()
    nw = info.num_cores * info.num_subcores
    D = emb.shape[1]
    b_per_w = count // nw
    mesh = plsc.VectorSubcoreMesh(core_axis_name="c", subcore_axis_name="s")

    @functools.partial(
        pl.kernel,
        mesh=mesh,
        out_type=jax.ShapeDtypeStruct((count, D), jnp.float32),
        scratch_types=[
            pltpu.VMEM((b_per_w,), jnp.int32),
            pltpu.VMEM((b_per_w, D), jnp.float32),
            pltpu.SemaphoreType.DMA,
        ],
    )
    def gather(emb_hbm, t_hbm, out_hbm, idx_v, rows_v, sem):
        wid = lax.axis_index("s") * info.num_cores + lax.axis_index("c")
        base = wid * b_per_w
        pltpu.sync_copy(t_hbm.at[pl.ds(offset + base, b_per_w)], idx_v)
        pltpu.async_copy(emb_hbm.at[idx_v], rows_v, sem).wait()
        pltpu.sync_copy(rows_v, out_hbm.at[pl.ds(base, b_per_w)])

    return gather(emb, t)


def _stream_body(e_ref, w_ref, b_ref, x_ref, out_ref, te_ref):
    @pl.when(pl.program_id(1) == 0)
    def _compute_te():
        te_ref[...] = (
            lax.dot_general(
                e_ref[...].astype(jnp.bfloat16), w_ref[...],
                (((1,), (1,)), ((), ())),
                preferred_element_type=jnp.float32,
            )
            + b_ref[...]
        )

    out_ref[...] = x_ref[...] + te_ref[...][None]


def _stream_body_aliased(e_ref, w_ref, b_ref, x_ref, tmp_ref, out_ref, te_ref):
    _stream_body(e_ref, w_ref, b_ref, x_ref, out_ref, te_ref)


_CHUNK = 128
_BHB = 16
_T0 = 256


def _stream_a(e0, Wb, b2, xr):
    bh, T, N = xr.shape
    return pl.pallas_call(
        _stream_body,
        grid=(_T0 // _CHUNK, bh // _BHB),
        in_specs=[
            pl.BlockSpec((_CHUNK, N), lambda i, j: (i, 0)),
            pl.BlockSpec((N, N), lambda i, j: (0, 0)),
            pl.BlockSpec((1, N), lambda i, j: (0, 0)),
            pl.BlockSpec((_BHB, _CHUNK, N), lambda i, j: (j, i, 0)),
        ],
        out_specs=pl.BlockSpec((_BHB, _CHUNK, N), lambda i, j: (j, i, 0)),
        out_shape=jax.ShapeDtypeStruct((bh, T, N), jnp.float32),
        scratch_shapes=[pltpu.VMEM((_CHUNK, N), jnp.float32)],
    )(e0, Wb, b2, xr)


def _stream_b(e1, Wb, b2, xr, tmp):
    bh, T, N = xr.shape
    s0 = _T0 // _CHUNK
    return pl.pallas_call(
        _stream_body_aliased,
        grid=((T - _T0) // _CHUNK, bh // _BHB),
        in_specs=[
            pl.BlockSpec((_CHUNK, N), lambda i, j: (i, 0)),
            pl.BlockSpec((N, N), lambda i, j: (0, 0)),
            pl.BlockSpec((1, N), lambda i, j: (0, 0)),
            pl.BlockSpec((_BHB, _CHUNK, N), lambda i, j: (j, i + s0, 0)),
            pl.BlockSpec(memory_space=pltpu.MemorySpace.HBM),
        ],
        out_specs=pl.BlockSpec((_BHB, _CHUNK, N), lambda i, j: (j, i + s0, 0)),
        out_shape=jax.ShapeDtypeStruct((bh, T, N), jnp.float32),
        scratch_shapes=[pltpu.VMEM((_CHUNK, N), jnp.float32)],
        input_output_aliases={4: 0},
    )(e1, Wb, b2, xr, tmp)


def kernel(x, t, emb, W, b):
    B2, H, T, N = x.shape
    bh = B2 * H
    xr = x.reshape(bh, T, N)
    e0 = _sc_gather(emb, t, 0, _T0)
    e1 = _sc_gather(emb, t, _T0, T - _T0)
    Wb = W.astype(jnp.bfloat16)
    b2 = b.reshape(1, N)
    tmp = _stream_a(e0, Wb, b2, xr)
    out = _stream_b(e1, Wb, b2, xr, tmp)
    return out.reshape(B2, H, T, N)
